# baseline (speedup 1.0000x reference)
import functools

import jax
import jax.numpy as jnp
from jax.experimental import pallas as pl
from jax.experimental.pallas import tpu as pltpu

D = 128


def _mlp_body(x_ref, wa_ref, ba_ref, wb_ref, bb_ref, o_ref):
    x = x_ref[...]
    h = x @ wa_ref[...] + ba_ref[...]
    h = h * jax.lax.logistic(h)
    o_ref[...] = h @ wb_ref[...] + bb_ref[...]


def _mlp(x, wa, ba, wb, bb, block=2048):
    n = x.shape[0]
    grid = (n // block,)
    return pl.pallas_call(
        _mlp_body,
        grid=grid,
        in_specs=[
            pl.BlockSpec((block, D), lambda i: (i, 0)),
            pl.BlockSpec((D, D), lambda i: (0, 0)),
            pl.BlockSpec((D,), lambda i: (0,)),
            pl.BlockSpec((D, D), lambda i: (0, 0)),
            pl.BlockSpec((D,), lambda i: (0,)),
        ],
        out_specs=pl.BlockSpec((block, D), lambda i: (i, 0)),
        out_shape=jax.ShapeDtypeStruct((n, D), jnp.float32),
    )(x, wa, ba, wb, bb)


def _final_body(a_ref, b_ref, c_ref, w0_ref, w1_ref, w2_ref, bias_ref, o_ref):
    h = (a_ref[...] @ w0_ref[...] + b_ref[...] @ w1_ref[...]
         + c_ref[...] @ w2_ref[...] + bias_ref[...])
    o_ref[...] = h * jax.lax.logistic(h)


def _final(h1, h2_up, h_down, W, b, block=2048):
    n = h1.shape[0]
    w0, w1, w2 = W[:D], W[D:2 * D], W[2 * D:]
    grid = (n // block,)
    mat = pl.BlockSpec((D, D), lambda i: (0, 0))
    blk = pl.BlockSpec((block, D), lambda i: (i, 0))
    return pl.pallas_call(
        _final_body,
        grid=grid,
        in_specs=[blk, blk, blk, mat, mat, mat, pl.BlockSpec((D,), lambda i: (0,))],
        out_specs=blk,
        out_shape=jax.ShapeDtypeStruct((n, D), jnp.float32),
    )(h1, h2_up, h_down, w0, w1, w2, b)


def kernel(feat, src12, dst12, src23, dst23, src34, dst34,
           W1a, b1a, W1b, b1b, W2a, b2a, W2b, b2b, W3a, b3a, W3b, b3b, W, b):
    N1, N2, N3, N4 = 100000, 50000, 25000, 12500

    h1 = _mlp(feat, W1a, b1a, W1b, b1b, block=2000)
    h2 = jax.ops.segment_prod(h1[src12], dst12, num_segments=N2)
    h2 = _mlp(h2, W2a, b2a, W2b, b2b, block=2000)
    h3 = jax.ops.segment_prod(h2[src23], dst23, num_segments=N3)
    h3 = _mlp(h3, W3a, b3a, W3b, b3b, block=1000)
    h4 = jax.ops.segment_prod(h3[src34], dst34, num_segments=N4)

    h2_up = jax.ops.segment_sum(h2[dst12], src12, num_segments=N1)
    h3_d = jax.ops.segment_sum(h4[dst34], src34, num_segments=N3)
    h2_d = jax.ops.segment_sum(h3_d[dst23], src23, num_segments=N2)
    h_down = jax.ops.segment_sum(h2_d[dst12], src12, num_segments=N1)

    return _final(h1, h2_up, h_down, W, b, block=2000)


# speedup vs baseline: 2.0181x; 2.0181x over previous
import functools

import jax
import jax.numpy as jnp
from jax import lax
from jax.experimental import pallas as pl
from jax.experimental.pallas import tpu as pltpu
from jax.experimental.pallas import tpu_sc as plsc

D = 128
N1, N2, N3, N4 = 100000, 50000, 25000, 12500
NLANES = 16
NW = 32
BATCH = 32


def _iota16():
    return lax.iota(jnp.int32, NLANES)


def _bucketize(srcp, dstp, sh, n_own, cap, sl):
    mesh = plsc.VectorSubcoreMesh(core_axis_name="c", subcore_axis_name="s")
    lmask = (1 << sh) - 1

    @functools.partial(
        pl.kernel, mesh=mesh,
        out_type=jax.ShapeDtypeStruct((NW * NW * cap,), jnp.int32),
        compiler_params=pltpu.CompilerParams(needs_layout_passes=False),
        scratch_types=[
            pltpu.VMEM((sl,), jnp.int32),
            pltpu.VMEM((sl,), jnp.int32),
            pltpu.VMEM((NW * cap,), jnp.int32),
            pltpu.SemaphoreType.DMA,
        ])
    def k(src_h, dst_h, bkt_h, src_v, dst_v, lists, sem):
        t = lax.axis_index("s") * 2 + lax.axis_index("c")
        pltpu.sync_copy(src_h.at[pl.ds(t * sl, sl)], src_v)
        pltpu.sync_copy(dst_h.at[pl.ds(t * sl, sl)], dst_v)

        sent = jnp.full((NLANES,), -1, jnp.int32)

        def fill(i, carry):
            lists[pl.ds(i * NLANES, NLANES)] = sent
            return carry
        lax.fori_loop(0, NW * cap // NLANES, fill, 0)

        iota = _iota16()

        def scan(i, cvs):
            cv0, cv1 = cvs
            sv = src_v[pl.ds(i * NLANES, NLANES)]
            dv = dst_v[pl.ds(i * NLANES, NLANES)]
            ow = lax.shift_right_logical(dv, sh)
            entry = jnp.bitwise_or(
                sv, lax.shift_left(jnp.bitwise_and(dv, lmask), 17))
            for o in range(n_own):
                m = ow == o
                inc = plsc.cumsum(m.astype(jnp.int32))
                last = inc[15]
                base = cv0[o] if o < 16 else cv1[o - 16]
                pos = (base + o * cap - 1) + inc
                plsc.store_scatter(lists, [pos], entry, mask=m)
                upd = jnp.where(iota == (o % 16), last, 0)
                if o < 16:
                    cv0 = cv0 + upd
                else:
                    cv1 = cv1 + upd
            return cv0, cv1

        z = jnp.zeros((NLANES,), jnp.int32)
        lax.fori_loop(0, sl // NLANES, scan, (z, z))
        pltpu.sync_copy(lists, bkt_h.at[pl.ds(t * NW * cap, NW * cap)])

    return k(srcp, dstp)


def _accumulate(table, bkt, sh, n_own, cap, r_sub, dcap, w):
    mesh = plsc.VectorSubcoreMesh(core_axis_name="c", subcore_axis_name="s")
    nsub = (1 << sh) // r_sub
    lsub = r_sub.bit_length() - 1
    out_rows = n_own << sh
    ncg = w // NLANES

    zrows = 64

    @functools.partial(
        pl.kernel, mesh=mesh,
        out_type=jax.ShapeDtypeStruct((out_rows * w,), jnp.float32),
        compiler_params=pltpu.CompilerParams(needs_layout_passes=False),
        scratch_types=[
            pltpu.VMEM((NW * cap,), jnp.int32),
            pltpu.VMEM((dcap + 4 * BATCH,), jnp.int32),
            pltpu.VMEM((BATCH,), jnp.int32),
            pltpu.VMEM((BATCH,), jnp.int32),
            pltpu.VMEM((BATCH, w), jnp.float32),
            pltpu.VMEM((BATCH, w), jnp.float32),
            pltpu.VMEM((r_sub * w,), jnp.float32),
            pltpu.SemaphoreType.DMA,
            pltpu.SemaphoreType.DMA,
            pltpu.SemaphoreType.DMA,
        ])
    def k(table_h, bkt_h, z_h, out_h,
          mybkt, dlst, gixa, gixb, rowsa, rowsb, acc, sems, sema, semb):
        t = lax.axis_index("s") * 2 + lax.axis_index("c")

        @pl.when(t < n_own)
        def _body():
            for p in range(NW):
                pltpu.async_copy(
                    bkt_h.at[pl.ds((p * NW + t) * cap, cap)],
                    mybkt.at[pl.ds(p * cap, cap)], sems)
            for p in range(NW):
                pltpu.make_async_copy(
                    bkt_h.at[pl.ds(0, cap)],
                    mybkt.at[pl.ds(0, cap)], sems).wait()

            zf = jnp.zeros((NLANES,), jnp.float32)
            zi = jnp.zeros((NLANES,), jnp.int32)

            def clr(i, carry):
                dlst[pl.ds(i * NLANES, NLANES)] = zi
                return carry
            lax.fori_loop(0, (dcap + 4 * BATCH) // NLANES, clr, 0)

            def subbody(sub, carry):
                for zk in range(r_sub // zrows):
                    pltpu.async_copy(
                        z_h, acc.at[pl.ds(zk * zrows * w, zrows * w)], sems)
                for zk in range(r_sub // zrows):
                    pltpu.make_async_copy(
                        z_h, acc.at[pl.ds(0, zrows * w)], sems).wait()

                nd = 0
                for p in range(NW):
                    def scan(i, cnt, p=p):
                        ev = mybkt[pl.ds(p * cap + i * NLANES, NLANES)]
                        m = (ev >= 0) & (
                            lax.shift_right_logical(ev, 17 + lsub) == sub)
                        inc = plsc.cumsum(m.astype(jnp.int32))
                        pos = cnt - 1 + inc
                        plsc.store_scatter(dlst, [pos], ev, mask=m)
                        return cnt + inc[15]
                    nd = lax.fori_loop(0, cap // NLANES, scan, nd)

                nbat = lax.shift_right_logical(nd + BATCH - 1, 5)
                nb2 = lax.shift_right_logical(nbat + 1, 1)
                hi = jnp.maximum(nbat - 1, 0)

                def fill_fire(bix, gix, rows, sem):
                    for g in range(BATCH // NLANES):
                        eg = dlst[pl.ds(bix * BATCH + g * NLANES, NLANES)]
                        gix[pl.ds(g * NLANES, NLANES)] = (
                            jnp.bitwise_and(eg, 0x1FFFF))
                    pltpu.async_copy(table_h.at[gix], rows, sem)

                fill_fire(jnp.minimum(0, hi), gixa, rowsa, sema)
                fill_fire(jnp.minimum(1, hi), gixb, rowsb, semb)

                iota = _iota16()

                def drain(b, rows):
                    for g in range(BATCH // NLANES):
                        eg = dlst[pl.ds(b * BATCH + g * NLANES, NLANES)]
                        for l in range(NLANES):
                            e = eg[l]
                            r = jnp.bitwise_and(
                                lax.shift_right_logical(e, 17), r_sub - 1)
                            valid = (b * BATCH + g * NLANES + l) < nd

                            @pl.when(valid)
                            def _add(g=g, l=l, r=r, rows=rows):
                                rowl = jnp.full(
                                    (NLANES,), g * NLANES + l, jnp.int32)
                                for cg in range(ncg):
                                    v = plsc.load_gather(
                                        rows, [rowl, cg * NLANES + iota])
                                    plsc.addupdate(
                                        acc.at[pl.ds(r * w + cg * NLANES,
                                                     NLANES)], v)

                def step(g2, carry):
                    ba = 2 * g2
                    pltpu.make_async_copy(
                        table_h.at[pl.ds(0, BATCH)], rowsa, sema).wait()
                    drain(ba, rowsa)
                    fill_fire(jnp.minimum(ba + 2, hi), gixa, rowsa, sema)
                    pltpu.make_async_copy(
                        table_h.at[pl.ds(0, BATCH)], rowsb, semb).wait()
                    drain(ba + 1, rowsb)
                    fill_fire(jnp.minimum(ba + 3, hi), gixb, rowsb, semb)
                    return carry
                lax.fori_loop(0, nb2, step, 0)
                pltpu.make_async_copy(
                    table_h.at[pl.ds(0, BATCH)], rowsa, sema).wait()
                pltpu.make_async_copy(
                    table_h.at[pl.ds(0, BATCH)], rowsb, semb).wait()

                pltpu.sync_copy(
                    acc,
                    out_h.at[pl.ds(((t << sh) + sub * r_sub) * w,
                                   r_sub * w)])
                return carry
            lax.fori_loop(0, nsub, subbody, 0)

    zeros = jnp.zeros((zrows * w,), jnp.float32)
    return k(table, bkt, zeros).reshape(out_rows, w)


def _seg_sum(table, src, dst, n_out):
    e = src.shape[0]
    w = table.shape[1]
    sh = max(9, (n_out - 1).bit_length() - 5)
    n_own = -(-(n_out + 1) // (1 << sh))
    r_sub = 256 if w > 128 else 512
    sl = -(-e // (NW * NLANES)) * NLANES
    e_pad = sl * NW
    cap = (int(e / (NW * n_own) * 1.8) // NLANES + 2) * NLANES
    dcap = (int(e / n_own / ((1 << sh) // r_sub) * 2.0) // 128 + 2) * 128
    srcp = jnp.pad(src, (0, e_pad - e), constant_values=0)
    dstp = jnp.pad(dst, (0, e_pad - e), constant_values=n_out)
    bkt = _bucketize(srcp, dstp, sh, n_own, cap, sl)
    return _accumulate(table, bkt, sh, n_own, cap, r_sub, dcap, w)


def _recon(blk):
    la = blk[:, :D]
    nc = blk[:, D:]
    par = nc - 2.0 * jnp.floor(nc * 0.5)
    return (1.0 - 2.0 * par) * jnp.exp(la)


def _logtab(h):
    return jnp.concatenate(
        [jnp.log(jnp.abs(h)), (h < 0).astype(jnp.float32)], axis=1)


def _stage1_body(x_ref, wa, ba, wb, bb, h_ref, t_ref):
    z = x_ref[...] @ wa[...] + ba[...]
    z = z * lax.logistic(z)
    h = z @ wb[...] + bb[...]
    h_ref[...] = h
    t_ref[...] = _logtab(h)


def _stage_mid_body(a_ref, wa, ba, wb, bb, h_ref, t_ref):
    p = _recon(a_ref[...])
    z = p @ wa[...] + ba[...]
    z = z * lax.logistic(z)
    h = z @ wb[...] + bb[...]
    h_ref[...] = h
    t_ref[...] = _logtab(h)


def _stage4_body(a_ref, h_ref):
    h_ref[...] = _recon(a_ref[...])


def _final_body(h1_ref, du_ref, w0, w1, w2, bias, o_ref):
    du = du_ref[...]
    h = (h1_ref[...] @ w0[...] + du[:, :D] @ w1[...]
         + du[:, D:] @ w2[...] + bias[...])
    o_ref[...] = h * lax.logistic(h)


def _mat_spec():
    return pl.BlockSpec((D, D), lambda i: (0, 0))


def _vec_spec():
    return pl.BlockSpec((D,), lambda i: (0,))


def _stage1(x, wa, ba, wb, bb, block=2000):
    n = x.shape[0]
    return pl.pallas_call(
        _stage1_body,
        grid=(n // block,),
        in_specs=[pl.BlockSpec((block, D), lambda i: (i, 0)),
                  _mat_spec(), _vec_spec(), _mat_spec(), _vec_spec()],
        out_specs=[pl.BlockSpec((block, D), lambda i: (i, 0)),
                   pl.BlockSpec((block, 2 * D), lambda i: (i, 0))],
        out_shape=[jax.ShapeDtypeStruct((n, D), jnp.float32),
                   jax.ShapeDtypeStruct((n, 2 * D), jnp.float32)],
    )(x, wa, ba, wb, bb)


def _stage_mid(a, wa, ba, wb, bb, n, block):
    return pl.pallas_call(
        _stage_mid_body,
        grid=(n // block,),
        in_specs=[pl.BlockSpec((block, 2 * D), lambda i: (i, 0)),
                  _mat_spec(), _vec_spec(), _mat_spec(), _vec_spec()],
        out_specs=[pl.BlockSpec((block, D), lambda i: (i, 0)),
                   pl.BlockSpec((block, 2 * D), lambda i: (i, 0))],
        out_shape=[jax.ShapeDtypeStruct((n, D), jnp.float32),
                   jax.ShapeDtypeStruct((n, 2 * D), jnp.float32)],
    )(a, wa, ba, wb, bb)


def _stage4(a, n, block):
    return pl.pallas_call(
        _stage4_body,
        grid=(n // block,),
        in_specs=[pl.BlockSpec((block, 2 * D), lambda i: (i, 0))],
        out_specs=pl.BlockSpec((block, D), lambda i: (i, 0)),
        out_shape=jax.ShapeDtypeStruct((n, D), jnp.float32),
    )(a)


def _final(h1, du, W, b, block=2000):
    n = h1.shape[0]
    w0, w1, w2 = W[:D], W[D:2 * D], W[2 * D:]
    return pl.pallas_call(
        _final_body,
        grid=(n // block,),
        in_specs=[pl.BlockSpec((block, D), lambda i: (i, 0)),
                  pl.BlockSpec((block, 2 * D), lambda i: (i, 0)),
                  _mat_spec(), _mat_spec(), _mat_spec(), _vec_spec()],
        out_specs=pl.BlockSpec((block, D), lambda i: (i, 0)),
        out_shape=jax.ShapeDtypeStruct((n, D), jnp.float32),
    )(h1, du, w0, w1, w2, b)


def kernel(feat, src12, dst12, src23, dst23, src34, dst34,
           W1a, b1a, W1b, b1b, W2a, b2a, W2b, b2b, W3a, b3a, W3b, b3b, W, b):
    h1, t1 = _stage1(feat, W1a, b1a, W1b, b1b)
    a12 = _seg_sum(t1, src12, dst12, N2)
    h2, t2 = _stage_mid(a12, W2a, b2a, W2b, b2b, N2, 2000)
    a23 = _seg_sum(t2, src23, dst23, N3)
    h3, t3 = _stage_mid(a23, W3a, b3a, W3b, b3b, N3, 1000)
    a34 = _seg_sum(t3, src34, dst34, N4)
    h4 = _stage4(a34, a34.shape[0], block=1600)

    a43 = _seg_sum(h4, dst34, src34, N3)
    a32 = _seg_sum(a43, dst23, src23, N2)
    tbl21 = jnp.concatenate([h2, a32[:N2]], axis=1)
    a21 = _seg_sum(tbl21, dst12, src12, N1)

    return _final(h1, a21, W, b)
